# baseline (speedup 1.0000x reference)
import functools

import jax
import jax.numpy as jnp
from jax import lax
from jax.experimental import pallas as pl
from jax.experimental.pallas import tpu as pltpu
from jax.experimental.pallas import tpu_sc as plsc

_VOCAB = 1000000
_DIM = 64
_B = 16384
_L = 16

_info = plsc.get_sparse_core_info()
_NC = _info.num_cores
_NS = _info.num_subcores
_NW = _NC * _NS
_BPW = _B // _NW
_CH = 128
_NCH = _BPW // _CH

_mesh = plsc.VectorSubcoreMesh(core_axis_name="c", subcore_axis_name="s")

_GATHER_DN = lax.GatherDimensionNumbers(
    offset_dims=(), collapsed_slice_dims=(0,), start_index_map=(0,))


def _shuffle(x, perm):
    return lax.gather(x, perm[:, None], _GATHER_DN, slice_sizes=(1,),
                      mode=lax.GatherScatterMode.PROMISE_IN_BOUNDS)


@functools.partial(
    pl.kernel,
    mesh=_mesh,
    compiler_params=pltpu.CompilerParams(use_tc_tiling_on_sc=False),
    out_type=jax.ShapeDtypeStruct((_B,), jnp.float32),
    scratch_types=[
        pltpu.VMEM((_BPW,), jnp.int32),
        pltpu.VMEM((_BPW,), jnp.int32),
        pltpu.VMEM((_BPW, _DIM), jnp.float32),
        pltpu.VMEM((_BPW, _DIM), jnp.float32),
        pltpu.VMEM((_BPW,), jnp.float32),
        pltpu.SemaphoreType.DMA,
    ],
)
def _sim_kernel(cidx_hbm, tidx_hbm, emb_hbm, sim_hbm,
                cidx_v, tidx_v, crows_v, trows_v, sim_v, sem):
    wid = lax.axis_index("s") * _NC + lax.axis_index("c")
    base = wid * _BPW

    pltpu.sync_copy(cidx_hbm.at[pl.ds(base, _BPW)], cidx_v)
    pltpu.sync_copy(tidx_hbm.at[pl.ds(base, _BPW)], tidx_v)

    copies = []
    for k in range(_NCH):
        copies.append(pltpu.async_copy(
            emb_hbm.at[cidx_v.at[pl.ds(k * _CH, _CH)]],
            crows_v.at[pl.ds(k * _CH, _CH), :], sem))
        copies.append(pltpu.async_copy(
            emb_hbm.at[tidx_v.at[pl.ds(k * _CH, _CH)]],
            trows_v.at[pl.ds(k * _CH, _CH), :], sem))
    for cp in copies:
        cp.wait()

    iota = jnp.arange(_L, dtype=jnp.int32)
    perms = [iota ^ h for h in (8, 4, 2, 1)]

    def group_body(g, carry):
        sim_g = jnp.zeros((_L,), jnp.float32)
        for j in range(_L):
            r = g * _L + j
            p = crows_v[r, pl.ds(0, _L)] * trows_v[r, pl.ds(0, _L)]
            for k in range(1, _DIM // _L):
                p = p + crows_v[r, pl.ds(k * _L, _L)] * trows_v[r, pl.ds(k * _L, _L)]
            for perm in perms:
                p = p + _shuffle(p, perm)
            sim_g = jnp.where(iota == j, p, sim_g)
        sim_v[pl.ds(g * _L, _L)] = sim_g
        return carry

    lax.fori_loop(0, _BPW // _L, group_body, 0)

    pltpu.sync_copy(sim_v, sim_hbm.at[pl.ds(base, _BPW)])


def _loss_body(sim_ref, label_ref, out_ref):
    s = sim_ref[...]
    lbl = label_ref[...]
    term = jnp.maximum(s, 0.0) - s * lbl + jnp.log1p(jnp.exp(-jnp.abs(s)))
    out_ref[0, 0] = jnp.sum(term)


def kernel(center_idx, target_idx, label, emb_weight, out_emb_weight):
    del out_emb_weight
    sim = _sim_kernel(center_idx, target_idx, emb_weight)
    loss = pl.pallas_call(
        _loss_body,
        out_shape=jax.ShapeDtypeStruct((1, 1), jnp.float32),
        out_specs=pl.BlockSpec(memory_space=pltpu.SMEM),
    )(sim.reshape(128, 128), label.reshape(128, 128))
    return loss[0, 0]
